# baseline (speedup 1.0000x reference)
import jax
import jax.numpy as jnp
from jax.experimental import pallas as pl
from jax.experimental.pallas import tpu as pltpu

B = 4096
D_IN = 1024
D_HID = 2048
D_EMB = 128

BM = 512


def _body(xq_ref, xd_ref, wq1_ref, bq1_ref, wq2_ref, bq2_ref,
          wd1_ref, bd1_ref, wd2_ref, bd2_ref, oq_ref, od_ref):
    hq = jnp.maximum(
        jnp.dot(xq_ref[:], wq1_ref[:], preferred_element_type=jnp.float32)
        + bq1_ref[:], 0.0)
    oq_ref[:] = (jnp.dot(hq.astype(jnp.bfloat16), wq2_ref[:],
                         preferred_element_type=jnp.float32)
                 + bq2_ref[:])
    hd = jnp.maximum(
        jnp.dot(xd_ref[:], wd1_ref[:], preferred_element_type=jnp.float32)
        + bd1_ref[:], 0.0)
    od_ref[:] = (jnp.dot(hd.astype(jnp.bfloat16), wd2_ref[:],
                         preferred_element_type=jnp.float32)
                 + bd2_ref[:])


def kernel(query, document, Wq1, bq1, Wq2, bq2, Wd1, bd1, Wd2, bd2):
    query = query.astype(jnp.bfloat16)
    document = document.astype(jnp.bfloat16)
    Wq1 = Wq1.astype(jnp.bfloat16)
    Wd1 = Wd1.astype(jnp.bfloat16)
    Wq2 = Wq2.astype(jnp.bfloat16)
    Wd2 = Wd2.astype(jnp.bfloat16)
    bq1_2d = bq1.reshape(1, D_HID)
    bq2_2d = bq2.reshape(1, D_EMB)
    bd1_2d = bd1.reshape(1, D_HID)
    bd2_2d = bd2.reshape(1, D_EMB)

    x_spec = pl.BlockSpec((BM, D_IN), lambda i: (i, 0))
    w1_spec = pl.BlockSpec((D_IN, D_HID), lambda i: (0, 0))
    b1_spec = pl.BlockSpec((1, D_HID), lambda i: (0, 0))
    w2_spec = pl.BlockSpec((D_HID, D_EMB), lambda i: (0, 0))
    b2_spec = pl.BlockSpec((1, D_EMB), lambda i: (0, 0))
    o_spec = pl.BlockSpec((BM, D_EMB), lambda i: (i, 0))

    oq, od = pl.pallas_call(
        _body,
        grid=(B // BM,),
        in_specs=[x_spec, x_spec,
                  w1_spec, b1_spec, w2_spec, b2_spec,
                  w1_spec, b1_spec, w2_spec, b2_spec],
        out_specs=[o_spec, o_spec],
        out_shape=[jax.ShapeDtypeStruct((B, D_EMB), jnp.float32),
                   jax.ShapeDtypeStruct((B, D_EMB), jnp.float32)],
        compiler_params=pltpu.CompilerParams(
            dimension_semantics=("arbitrary",),
        ),
    )(query, document, Wq1, bq1_2d, Wq2, bq2_2d, Wd1, bd1_2d, Wd2, bd2_2d)
    return (oq, od)


# speedup vs baseline: 1.5341x; 1.5341x over previous
import jax
import jax.numpy as jnp
from jax.experimental import pallas as pl
from jax.experimental.pallas import tpu as pltpu

B = 4096
D_IN = 1024
D_HID = 2048
D_EMB = 128

BM = 1024


def _body(xq_ref, xd_ref, wq1_ref, bq1_ref, wq2_ref, bq2_ref,
          wd1_ref, bd1_ref, wd2_ref, bd2_ref, oq_ref, od_ref):
    hq = jnp.maximum(
        jnp.dot(xq_ref[:], wq1_ref[:], preferred_element_type=jnp.float32)
        + bq1_ref[:], 0.0)
    oq_ref[:] = (jnp.dot(hq.astype(jnp.bfloat16), wq2_ref[:],
                         preferred_element_type=jnp.float32)
                 + bq2_ref[:])
    hd = jnp.maximum(
        jnp.dot(xd_ref[:], wd1_ref[:], preferred_element_type=jnp.float32)
        + bd1_ref[:], 0.0)
    od_ref[:] = (jnp.dot(hd.astype(jnp.bfloat16), wd2_ref[:],
                         preferred_element_type=jnp.float32)
                 + bd2_ref[:])


def kernel(query, document, Wq1, bq1, Wq2, bq2, Wd1, bd1, Wd2, bd2):
    bq1_2d = bq1.reshape(1, D_HID)
    bq2_2d = bq2.reshape(1, D_EMB)
    bd1_2d = bd1.reshape(1, D_HID)
    bd2_2d = bd2.reshape(1, D_EMB)

    x_spec = pl.BlockSpec((BM, D_IN), lambda i: (i, 0))
    w1_spec = pl.BlockSpec((D_IN, D_HID), lambda i: (0, 0))
    b1_spec = pl.BlockSpec((1, D_HID), lambda i: (0, 0))
    w2_spec = pl.BlockSpec((D_HID, D_EMB), lambda i: (0, 0))
    b2_spec = pl.BlockSpec((1, D_EMB), lambda i: (0, 0))
    o_spec = pl.BlockSpec((BM, D_EMB), lambda i: (i, 0))

    oq, od = pl.pallas_call(
        _body,
        grid=(B // BM,),
        in_specs=[x_spec, x_spec,
                  w1_spec, b1_spec, w2_spec, b2_spec,
                  w1_spec, b1_spec, w2_spec, b2_spec],
        out_specs=[o_spec, o_spec],
        out_shape=[jax.ShapeDtypeStruct((B, D_EMB), jnp.float32),
                   jax.ShapeDtypeStruct((B, D_EMB), jnp.float32)],
        compiler_params=pltpu.CompilerParams(
            dimension_semantics=("arbitrary",),
        ),
    )(query, document, Wq1, bq1_2d, Wq2, bq2_2d, Wd1, bd1_2d, Wd2, bd2_2d)
    return (oq, od)
